# baseline (device time: 199655 ns/iter reference)
import jax
import jax.numpy as jnp
from jax import lax
from jax.experimental import pallas as pl
from jax.experimental.pallas import tpu as pltpu

N_DEV = 8


def kernel(x, w_mat, scale_x, scale_w):
    m_per, k = x.shape
    _, n_per = w_mat.shape

    x8 = x.astype(jnp.float8_e4m3fn)
    wb = w_mat.astype(jnp.bfloat16)
    s = (scale_x[0] * scale_w[0]).reshape(1, 1)

    def body(x_ref, w_ref, s_ref, out_ref, comm_ref, send_sems, recv_sems):
        my = lax.axis_index("i")
        left = lax.rem(my + (N_DEV - 1), N_DEV)
        right = lax.rem(my + 1, N_DEV)

        barrier_sem = pltpu.get_barrier_semaphore()
        for nbr in (left, right):
            pl.semaphore_signal(
                barrier_sem, inc=1,
                device_id=(nbr,), device_id_type=pl.DeviceIdType.MESH,
            )
        pl.semaphore_wait(barrier_sem, 2)

        scale = s_ref[0, 0]

        def gemm_store(chunk, origin):
            acc = jnp.dot(
                chunk.astype(jnp.bfloat16), w_ref[...],
                preferred_element_type=jnp.float32,
            )
            out_ref[pl.ds(origin * m_per, m_per), :] = jnp.maximum(acc * scale, 0.0)

        gemm_store(x_ref[...], my)

        for h in range(N_DEV - 1):
            src = x_ref if h == 0 else comm_ref.at[h - 1]
            rdma = pltpu.make_async_remote_copy(
                src_ref=src,
                dst_ref=comm_ref.at[h],
                send_sem=send_sems.at[h],
                recv_sem=recv_sems.at[h],
                device_id=(right,),
                device_id_type=pl.DeviceIdType.MESH,
            )
            rdma.start()
            rdma.wait()
            origin = lax.rem(my + (2 * N_DEV - 1 - h), N_DEV)
            gemm_store(comm_ref[h], origin)

    return pl.pallas_call(
        body,
        out_shape=jax.ShapeDtypeStruct((N_DEV * m_per, n_per), jnp.float32),
        in_specs=[
            pl.BlockSpec(memory_space=pltpu.VMEM),
            pl.BlockSpec(memory_space=pltpu.VMEM),
            pl.BlockSpec(memory_space=pltpu.SMEM),
        ],
        out_specs=pl.BlockSpec(memory_space=pltpu.VMEM),
        scratch_shapes=[
            pltpu.VMEM((N_DEV - 1, m_per, k), jnp.float8_e4m3fn),
            pltpu.SemaphoreType.DMA((N_DEV - 1,)),
            pltpu.SemaphoreType.DMA((N_DEV - 1,)),
        ],
        compiler_params=pltpu.CompilerParams(collective_id=0),
    )(x8, wb, s)


# device time: 106130 ns/iter; 1.8812x vs baseline; 1.8812x over previous
import jax
import jax.numpy as jnp
from jax import lax
from jax.experimental import pallas as pl
from jax.experimental.pallas import tpu as pltpu

N_DEV = 8


def kernel(x, w_mat, scale_x, scale_w):
    m_per, k = x.shape
    _, n_per = w_mat.shape
    half = m_per // 2

    x8 = x.astype(jnp.float8_e4m3fn)
    wb = w_mat.astype(jnp.bfloat16)
    s = (scale_x[0] * scale_w[0]).reshape(1, 1)

    def body(x_ref, w_ref, s_ref, out_ref,
             comm_r, comm_l, send_r, recv_r, send_l, recv_l):
        my = lax.axis_index("i")
        left = lax.rem(my + (N_DEV - 1), N_DEV)
        right = lax.rem(my + 1, N_DEV)

        barrier_sem = pltpu.get_barrier_semaphore()
        for nbr in (left, right):
            pl.semaphore_signal(
                barrier_sem, inc=1,
                device_id=(nbr,), device_id_type=pl.DeviceIdType.MESH,
            )
        pl.semaphore_wait(barrier_sem, 2)

        scale = s_ref[0, 0]

        def gemm_store(chunk, row0):
            acc = jnp.dot(
                chunk.astype(jnp.bfloat16), w_ref[...],
                preferred_element_type=jnp.float32,
            )
            out_ref[pl.ds(row0, chunk.shape[0]), :] = jnp.maximum(acc * scale, 0.0)

        def compute_hop(h):
            orig_r = lax.rem(my + (2 * N_DEV - 1 - h), N_DEV)
            orig_l = lax.rem(my + 1 + h, N_DEV)
            gemm_store(comm_r[h], orig_r * m_per)
            gemm_store(comm_l[h], orig_l * m_per + half)

        rdmas = []
        for h in range(N_DEV - 1):
            src_r = x_ref.at[pl.ds(0, half), :] if h == 0 else comm_r.at[h - 1]
            src_l = x_ref.at[pl.ds(half, half), :] if h == 0 else comm_l.at[h - 1]
            rdma_r = pltpu.make_async_remote_copy(
                src_ref=src_r, dst_ref=comm_r.at[h],
                send_sem=send_r.at[h], recv_sem=recv_r.at[h],
                device_id=(right,), device_id_type=pl.DeviceIdType.MESH,
            )
            rdma_l = pltpu.make_async_remote_copy(
                src_ref=src_l, dst_ref=comm_l.at[h],
                send_sem=send_l.at[h], recv_sem=recv_l.at[h],
                device_id=(left,), device_id_type=pl.DeviceIdType.MESH,
            )
            rdma_r.start()
            rdma_l.start()
            rdmas.append((rdma_r, rdma_l))
            if h == 0:
                gemm_store(x_ref[...], my * m_per)
            else:
                compute_hop(h - 1)
            rdma_r.wait_recv()
            rdma_l.wait_recv()
        compute_hop(N_DEV - 2)
        for rdma_r, rdma_l in rdmas:
            rdma_r.wait_send()
            rdma_l.wait_send()

    return pl.pallas_call(
        body,
        out_shape=jax.ShapeDtypeStruct((N_DEV * m_per, n_per), jnp.float32),
        in_specs=[
            pl.BlockSpec(memory_space=pltpu.VMEM),
            pl.BlockSpec(memory_space=pltpu.VMEM),
            pl.BlockSpec(memory_space=pltpu.SMEM),
        ],
        out_specs=pl.BlockSpec(memory_space=pltpu.VMEM),
        scratch_shapes=[
            pltpu.VMEM((N_DEV - 1, half, k), jnp.float8_e4m3fn),
            pltpu.VMEM((N_DEV - 1, half, k), jnp.float8_e4m3fn),
            pltpu.SemaphoreType.DMA((N_DEV - 1,)),
            pltpu.SemaphoreType.DMA((N_DEV - 1,)),
            pltpu.SemaphoreType.DMA((N_DEV - 1,)),
            pltpu.SemaphoreType.DMA((N_DEV - 1,)),
        ],
        compiler_params=pltpu.CompilerParams(collective_id=0),
    )(x8, wb, s)


# device time: 84515 ns/iter; 2.3624x vs baseline; 1.2558x over previous
import jax
import jax.numpy as jnp
from jax import lax
from jax.experimental import pallas as pl
from jax.experimental.pallas import tpu as pltpu

N_DEV = 8
N_PLANE = 4
N_HOP = N_PLANE - 1


def kernel(x, w_mat, scale_x, scale_w):
    m_per, k = x.shape
    _, n_per = w_mat.shape
    half = m_per // 2

    x8 = x.astype(jnp.float8_e4m3fn)
    wb = w_mat.astype(jnp.bfloat16)
    s = (scale_x[0] * scale_w[0]).reshape(1, 1)

    def body(x_ref, w_ref, s_ref, out_ref, zbuf,
             comm_ar, comm_al, comm_br, comm_bl,
             z_send, z_recv,
             ar_send, ar_recv, al_send, al_recv,
             br_send, br_recv, bl_send, bl_recv):
        my = lax.axis_index("i")
        zz = my // N_PLANE
        p = lax.rem(my, N_PLANE)
        right = zz * N_PLANE + lax.rem(p + 1, N_PLANE)
        left = zz * N_PLANE + lax.rem(p + 3, N_PLANE)
        zpartner = lax.rem(my + N_PLANE, N_DEV)

        barrier_sem = pltpu.get_barrier_semaphore()
        for nbr in (left, right, zpartner):
            pl.semaphore_signal(
                barrier_sem, inc=1,
                device_id=(nbr,), device_id_type=pl.DeviceIdType.MESH,
            )
        pl.semaphore_wait(barrier_sem, 3)

        scale = s_ref[0, 0]

        def gemm_store(chunk, row0):
            acc = jnp.dot(
                chunk.astype(jnp.bfloat16), w_ref[...],
                preferred_element_type=jnp.float32,
            )
            out_ref[pl.ds(row0, chunk.shape[0]), :] = jnp.maximum(acc * scale, 0.0)

        def ring_rdma(src, dst, send_s, recv_s, target):
            return pltpu.make_async_remote_copy(
                src_ref=src, dst_ref=dst, send_sem=send_s, recv_sem=recv_s,
                device_id=(target,), device_id_type=pl.DeviceIdType.MESH,
            )

        z_rdmas = [
            ring_rdma(x_ref.at[pl.ds(i * half, half), :],
                      zbuf.at[i],
                      z_send.at[i], z_recv.at[i], zpartner)
            for i in range(2)
        ]

        def hop_rdmas(h, wave):
            if wave == 0:
                cr, cl = comm_ar, comm_al
                sr = (ar_send, ar_recv, al_send, al_recv)
                src_r = x_ref.at[pl.ds(0, half), :] if h == 0 else comm_ar.at[h - 1]
                src_l = x_ref.at[pl.ds(half, half), :] if h == 0 else comm_al.at[h - 1]
            else:
                cr, cl = comm_br, comm_bl
                sr = (br_send, br_recv, bl_send, bl_recv)
                src_r = zbuf.at[0] if h == 0 else comm_br.at[h - 1]
                src_l = zbuf.at[1] if h == 0 else comm_bl.at[h - 1]
            r = ring_rdma(src_r, cr.at[h], sr[0].at[h], sr[1].at[h], right)
            l = ring_rdma(src_l, cl.at[h], sr[2].at[h], sr[3].at[h], left)
            return r, l

        def compute_hop(h, wave):
            zoff = (zz if wave == 0 else (1 - zz)) * N_PLANE
            orig_r = zoff + lax.rem(p + (N_PLANE - 1 - h), N_PLANE)
            orig_l = zoff + lax.rem(p + 1 + h, N_PLANE)
            cr = comm_ar if wave == 0 else comm_br
            cl = comm_al if wave == 0 else comm_bl
            gemm_store(cr[h], orig_r * m_per)
            gemm_store(cl[h], orig_l * m_per + half)

        for rd in z_rdmas:
            rd.start()
        started = list(z_rdmas)

        def start_hop(h, wave):
            rds = hop_rdmas(h, wave)
            for r in rds:
                r.start()
            started.extend(rds)
            return rds

        a0 = start_hop(0, 0)
        gemm_store(x_ref[...], my * m_per)

        for rd in a0:
            rd.wait_recv()
        a1 = start_hop(1, 0)
        compute_hop(0, 0)

        for rd in z_rdmas:
            rd.wait_recv()
        b0 = start_hop(0, 1)
        gemm_store(zbuf[0], zpartner * m_per)
        gemm_store(zbuf[1], zpartner * m_per + half)

        for rd in a1:
            rd.wait_recv()
        a2 = start_hop(2, 0)
        compute_hop(1, 0)

        for rd in b0:
            rd.wait_recv()
        b1 = start_hop(1, 1)
        compute_hop(0, 1)

        for rd in a2:
            rd.wait_recv()
        compute_hop(2, 0)

        for rd in b1:
            rd.wait_recv()
        b2 = start_hop(2, 1)
        compute_hop(1, 1)

        for rd in b2:
            rd.wait_recv()
        compute_hop(2, 1)

        for rd in started:
            rd.wait_send()

    hop_buf = lambda: pltpu.VMEM((N_HOP, half, k), jnp.float8_e4m3fn)
    hop_sem = lambda: pltpu.SemaphoreType.DMA((N_HOP,))
    return pl.pallas_call(
        body,
        out_shape=jax.ShapeDtypeStruct((N_DEV * m_per, n_per), jnp.float32),
        in_specs=[
            pl.BlockSpec(memory_space=pltpu.VMEM),
            pl.BlockSpec(memory_space=pltpu.VMEM),
            pl.BlockSpec(memory_space=pltpu.SMEM),
        ],
        out_specs=pl.BlockSpec(memory_space=pltpu.VMEM),
        scratch_shapes=[
            pltpu.VMEM((2, half, k), jnp.float8_e4m3fn),
            hop_buf(), hop_buf(), hop_buf(), hop_buf(),
            pltpu.SemaphoreType.DMA((2,)),
            pltpu.SemaphoreType.DMA((2,)),
            hop_sem(), hop_sem(), hop_sem(), hop_sem(),
            hop_sem(), hop_sem(), hop_sem(), hop_sem(),
        ],
        compiler_params=pltpu.CompilerParams(collective_id=0),
    )(x8, wb, s)


# device time: 73419 ns/iter; 2.7194x vs baseline; 1.1511x over previous
import jax
import jax.numpy as jnp
from jax import lax
from jax.experimental import pallas as pl
from jax.experimental.pallas import tpu as pltpu

N_DEV = 8
N_PLANE = 4
N_HOP = N_PLANE - 1


def kernel(x, w_mat, scale_x, scale_w):
    m_per, k = x.shape
    _, n_per = w_mat.shape
    half = m_per // 2

    x8 = x.astype(jnp.float8_e4m3fn)
    wb = w_mat.astype(jnp.bfloat16)
    s = (scale_x[0] * scale_w[0]).reshape(1, 1)

    def body(x_ref, w_ref, s_ref, out_ref,
             zbuf, comm_ar, comm_al, comm_br, comm_bl, czbuf,
             z_send, z_recv, ar_send, ar_recv, al_send, al_recv,
             br_send, br_recv, bl_send, bl_recv, c_send, c_recv):
        my = lax.axis_index("i")
        zz = my // N_PLANE
        p = lax.rem(my, N_PLANE)
        right = zz * N_PLANE + lax.rem(p + 1, N_PLANE)
        left = zz * N_PLANE + lax.rem(p + 3, N_PLANE)
        zpartner = lax.rem(my + N_PLANE, N_DEV)
        other = (1 - zz) * N_PLANE

        barrier_sem = pltpu.get_barrier_semaphore()
        for nbr in (left, right, zpartner):
            pl.semaphore_signal(
                barrier_sem, inc=1,
                device_id=(nbr,), device_id_type=pl.DeviceIdType.MESH,
            )
        pl.semaphore_wait(barrier_sem, 3)

        scale = s_ref[0, 0]

        def gemm_store(chunk, row0):
            acc = jnp.dot(
                chunk.astype(jnp.bfloat16), w_ref[...],
                preferred_element_type=jnp.float32,
            )
            out_ref[pl.ds(row0, chunk.shape[0]), :] = jnp.maximum(acc * scale, 0.0)

        def rdma(src, dst, send_s, recv_s, target):
            return pltpu.make_async_remote_copy(
                src_ref=src, dst_ref=dst, send_sem=send_s, recv_sem=recv_s,
                device_id=(target,), device_id_type=pl.DeviceIdType.MESH,
            )

        z_rdmas = [
            rdma(x_ref.at[pl.ds(i * half, half), :], zbuf.at[i],
                 z_send.at[i], z_recv.at[i], zpartner)
            for i in range(2)
        ]

        def a_hop(h):
            src_r = x_ref.at[pl.ds(0, half), :] if h == 0 else comm_ar.at[h - 1]
            src_l = x_ref.at[pl.ds(half, half), :] if h == 0 else comm_al.at[h - 1]
            return (
                rdma(src_r, comm_ar.at[h], ar_send.at[h], ar_recv.at[h], right),
                rdma(src_l, comm_al.at[h], al_send.at[h], al_recv.at[h], left),
            )

        def compute_a(h):
            orig_r = zz * N_PLANE + lax.rem(p + (N_PLANE - 1 - h), N_PLANE)
            orig_l = zz * N_PLANE + lax.rem(p + 1 + h, N_PLANE)
            gemm_store(comm_ar[h], orig_r * m_per)
            gemm_store(comm_al[h], orig_l * m_per + half)

        started = list(z_rdmas)
        for rd in z_rdmas:
            rd.start()

        def start(rds):
            for r in rds:
                r.start()
            started.extend(rds)
            return rds

        a0 = start(a_hop(0))
        gemm_store(x_ref[...], my * m_per)

        for rd in a0:
            rd.wait_recv()
        a1 = start(a_hop(1))
        compute_a(0)

        for rd in z_rdmas:
            rd.wait_recv()
        b = start([
            rdma(zbuf, comm_br, br_send, br_recv, right),
            rdma(zbuf, comm_bl, bl_send, bl_recv, left),
        ])
        gemm_store(zbuf[0], zpartner * m_per)
        gemm_store(zbuf[1], zpartner * m_per + half)

        for rd in a1:
            rd.wait_recv()
        a2 = start(a_hop(2))
        c = start([
            rdma(comm_ar.at[1], czbuf.at[0], c_send.at[0], c_recv.at[0], zpartner),
            rdma(comm_al.at[1], czbuf.at[1], c_send.at[1], c_recv.at[1], zpartner),
        ])
        compute_a(1)

        for rd in b:
            rd.wait_recv()
        orig_br = other + lax.rem(p + 3, N_PLANE)
        orig_bl = other + lax.rem(p + 1, N_PLANE)
        gemm_store(comm_br[0], orig_br * m_per)
        gemm_store(comm_br[1], orig_br * m_per + half)
        gemm_store(comm_bl[0], orig_bl * m_per)
        gemm_store(comm_bl[1], orig_bl * m_per + half)

        for rd in a2:
            rd.wait_recv()
        compute_a(2)

        for rd in c:
            rd.wait_recv()
        orig_c = other + lax.rem(p + 2, N_PLANE)
        gemm_store(czbuf[0], orig_c * m_per)
        gemm_store(czbuf[1], orig_c * m_per + half)

        for rd in started:
            rd.wait_send()

    half_pair = lambda: pltpu.VMEM((2, half, k), jnp.float8_e4m3fn)
    hop_buf = lambda: pltpu.VMEM((N_HOP, half, k), jnp.float8_e4m3fn)
    hop_sem = lambda: pltpu.SemaphoreType.DMA((N_HOP,))
    pair_sem = lambda: pltpu.SemaphoreType.DMA((2,))
    return pl.pallas_call(
        body,
        out_shape=jax.ShapeDtypeStruct((N_DEV * m_per, n_per), jnp.float32),
        in_specs=[
            pl.BlockSpec(memory_space=pltpu.VMEM),
            pl.BlockSpec(memory_space=pltpu.VMEM),
            pl.BlockSpec(memory_space=pltpu.SMEM),
        ],
        out_specs=pl.BlockSpec(memory_space=pltpu.VMEM),
        scratch_shapes=[
            half_pair(),
            hop_buf(), hop_buf(),
            half_pair(), half_pair(),
            half_pair(),
            pair_sem(), pair_sem(),
            hop_sem(), hop_sem(), hop_sem(), hop_sem(),
            pltpu.SemaphoreType.DMA, pltpu.SemaphoreType.DMA,
            pltpu.SemaphoreType.DMA, pltpu.SemaphoreType.DMA,
            pair_sem(), pair_sem(),
        ],
        compiler_params=pltpu.CompilerParams(collective_id=0),
    )(x8, wb, s)


# device time: 72818 ns/iter; 2.7418x vs baseline; 1.0083x over previous
import jax
import jax.numpy as jnp
from jax import lax
from jax.experimental import pallas as pl
from jax.experimental.pallas import tpu as pltpu

N_DEV = 8
N_PLANE = 4
N_HOP = N_PLANE - 1


def kernel(x, w_mat, scale_x, scale_w):
    m_per, k = x.shape
    _, n_per = w_mat.shape
    half = m_per // 2

    s = (scale_x[0] * scale_w[0]).reshape(1, 1)

    def body(x_ref, w_ref, s_ref, out_ref,
             xc, wbc, zbuf, comm_ar, comm_al, comm_br, comm_bl, czbuf,
             z_send, z_recv, ar_send, ar_recv, al_send, al_recv,
             br_send, br_recv, bl_send, bl_recv, c_send, c_recv):
        my = lax.axis_index("i")
        zz = my // N_PLANE
        p = lax.rem(my, N_PLANE)
        right = zz * N_PLANE + lax.rem(p + 1, N_PLANE)
        left = zz * N_PLANE + lax.rem(p + 3, N_PLANE)
        zpartner = lax.rem(my + N_PLANE, N_DEV)
        other = (1 - zz) * N_PLANE

        barrier_sem = pltpu.get_barrier_semaphore()
        for nbr in (left, right, zpartner):
            pl.semaphore_signal(
                barrier_sem, inc=1,
                device_id=(nbr,), device_id_type=pl.DeviceIdType.MESH,
            )
        pl.semaphore_wait(barrier_sem, 3)

        scale = s_ref[0, 0]

        def gemm_store(chunk, row0):
            acc = jnp.dot(
                chunk.astype(jnp.bfloat16), wbc[...],
                preferred_element_type=jnp.float32,
            )
            out_ref[pl.ds(row0, chunk.shape[0]), :] = jnp.maximum(acc * scale, 0.0)

        def rdma(src, dst, send_s, recv_s, target):
            return pltpu.make_async_remote_copy(
                src_ref=src, dst_ref=dst, send_sem=send_s, recv_sem=recv_s,
                device_id=(target,), device_id_type=pl.DeviceIdType.MESH,
            )

        xc[0] = x_ref[pl.ds(0, half), :].astype(jnp.float8_e4m3fn)
        xc[1] = x_ref[pl.ds(half, half), :].astype(jnp.float8_e4m3fn)

        z_rdmas = [
            rdma(xc.at[i], zbuf.at[i],
                 z_send.at[i], z_recv.at[i], zpartner)
            for i in range(2)
        ]

        def a_hop(h):
            src_r = xc.at[0] if h == 0 else comm_ar.at[h - 1]
            src_l = xc.at[1] if h == 0 else comm_al.at[h - 1]
            return (
                rdma(src_r, comm_ar.at[h], ar_send.at[h], ar_recv.at[h], right),
                rdma(src_l, comm_al.at[h], al_send.at[h], al_recv.at[h], left),
            )

        def compute_a(h):
            orig_r = zz * N_PLANE + lax.rem(p + (N_PLANE - 1 - h), N_PLANE)
            orig_l = zz * N_PLANE + lax.rem(p + 1 + h, N_PLANE)
            gemm_store(comm_ar[h], orig_r * m_per)
            gemm_store(comm_al[h], orig_l * m_per + half)

        started = list(z_rdmas)
        for rd in z_rdmas:
            rd.start()

        def start(rds):
            for r in rds:
                r.start()
            started.extend(rds)
            return rds

        a0 = start(a_hop(0))
        wbc[...] = w_ref[...].astype(jnp.bfloat16)
        gemm_store(x_ref[...], my * m_per)

        for rd in a0:
            rd.wait_recv()
        a1 = start(a_hop(1))
        compute_a(0)

        for rd in z_rdmas:
            rd.wait_recv()
        b = start([
            rdma(zbuf, comm_br, br_send, br_recv, right),
            rdma(zbuf, comm_bl, bl_send, bl_recv, left),
        ])
        gemm_store(zbuf[0], zpartner * m_per)
        gemm_store(zbuf[1], zpartner * m_per + half)

        for rd in a1:
            rd.wait_recv()
        a2 = start(a_hop(2))
        c = start([
            rdma(comm_ar.at[1], czbuf.at[0], c_send.at[0], c_recv.at[0], zpartner),
            rdma(comm_al.at[1], czbuf.at[1], c_send.at[1], c_recv.at[1], zpartner),
        ])
        compute_a(1)

        for rd in b:
            rd.wait_recv()
        orig_br = other + lax.rem(p + 3, N_PLANE)
        orig_bl = other + lax.rem(p + 1, N_PLANE)
        gemm_store(comm_br[0], orig_br * m_per)
        gemm_store(comm_br[1], orig_br * m_per + half)
        gemm_store(comm_bl[0], orig_bl * m_per)
        gemm_store(comm_bl[1], orig_bl * m_per + half)

        for rd in c:
            rd.wait_recv()
        orig_c = other + lax.rem(p + 2, N_PLANE)
        gemm_store(czbuf[0], orig_c * m_per)
        gemm_store(czbuf[1], orig_c * m_per + half)

        for rd in a2:
            rd.wait_recv()
        compute_a(2)

        for rd in started:
            rd.wait_send()

    half_pair = lambda: pltpu.VMEM((2, half, k), jnp.float8_e4m3fn)
    hop_buf = lambda: pltpu.VMEM((N_HOP, half, k), jnp.float8_e4m3fn)
    hop_sem = lambda: pltpu.SemaphoreType.DMA((N_HOP,))
    pair_sem = lambda: pltpu.SemaphoreType.DMA((2,))
    return pl.pallas_call(
        body,
        out_shape=jax.ShapeDtypeStruct((N_DEV * m_per, n_per), jnp.float32),
        in_specs=[
            pl.BlockSpec(memory_space=pltpu.VMEM),
            pl.BlockSpec(memory_space=pltpu.VMEM),
            pl.BlockSpec(memory_space=pltpu.SMEM),
        ],
        out_specs=pl.BlockSpec(memory_space=pltpu.VMEM),
        scratch_shapes=[
            half_pair(),
            pltpu.VMEM((k, n_per), jnp.bfloat16),
            half_pair(),
            hop_buf(), hop_buf(),
            half_pair(), half_pair(),
            half_pair(),
            pair_sem(), pair_sem(),
            hop_sem(), hop_sem(), hop_sem(), hop_sem(),
            pltpu.SemaphoreType.DMA, pltpu.SemaphoreType.DMA,
            pltpu.SemaphoreType.DMA, pltpu.SemaphoreType.DMA,
            pair_sem(), pair_sem(),
        ],
        compiler_params=pltpu.CompilerParams(collective_id=0),
    )(x, w_mat, s)


# device time: 70395 ns/iter; 2.8362x vs baseline; 1.0344x over previous
import jax
import jax.numpy as jnp
from jax import lax
from jax.experimental import pallas as pl
from jax.experimental.pallas import tpu as pltpu

N_DEV = 8
N_PLANE = 4
N_HOP = N_PLANE - 1
DELTA = 64


def kernel(x, w_mat, scale_x, scale_w):
    m_per, k = x.shape
    _, n_per = w_mat.shape
    half = m_per // 2
    rest = half - DELTA

    s = (scale_x[0] * scale_w[0]).reshape(1, 1)

    def body(x_ref, w_ref, s_ref, out_ref,
             xc, wbc, zbuf, comm_ar, comm_al,
             br_a, br_b, bl_a, bl_b, czbuf, czb2r, czb2l,
             st_br, st_bl, st_c2r, st_c2l,
             z_send, z_recv, ar_send, ar_recv, al_send, al_recv,
             br_send, br_recv, bl_send, bl_recv,
             c_send, c_recv, c2_send, c2_recv):
        my = lax.axis_index("i")
        zz = my // N_PLANE
        p = lax.rem(my, N_PLANE)
        right = zz * N_PLANE + lax.rem(p + 1, N_PLANE)
        left = zz * N_PLANE + lax.rem(p + 3, N_PLANE)
        zpartner = lax.rem(my + N_PLANE, N_DEV)
        other = (1 - zz) * N_PLANE

        barrier_sem = pltpu.get_barrier_semaphore()
        for nbr in (left, right, zpartner):
            pl.semaphore_signal(
                barrier_sem, inc=1,
                device_id=(nbr,), device_id_type=pl.DeviceIdType.MESH,
            )
        pl.semaphore_wait(barrier_sem, 3)

        scale = s_ref[0, 0]

        def gemm_store(chunk, row0):
            acc = jnp.dot(
                chunk.astype(jnp.bfloat16), wbc[...],
                preferred_element_type=jnp.float32,
            )
            out_ref[pl.ds(row0, chunk.shape[0]), :] = jnp.maximum(acc * scale, 0.0)

        def rdma(src, dst, send_s, recv_s, target):
            return pltpu.make_async_remote_copy(
                src_ref=src, dst_ref=dst, send_sem=send_s, recv_sem=recv_s,
                device_id=(target,), device_id_type=pl.DeviceIdType.MESH,
            )

        def a_hop(h):
            src_r = xc.at[0] if h == 0 else comm_ar.at[h - 1]
            src_l = xc.at[1] if h == 0 else comm_al.at[h - 1]
            return (
                rdma(src_r, comm_ar.at[h], ar_send.at[h], ar_recv.at[h], right),
                rdma(src_l, comm_al.at[h], al_send.at[h], al_recv.at[h], left),
            )

        def compute_a(h):
            orig_r = zz * N_PLANE + lax.rem(p + (N_PLANE - 1 - h), N_PLANE)
            orig_l = zz * N_PLANE + lax.rem(p + 1 + h, N_PLANE)
            gemm_store(comm_ar[h], orig_r * m_per)
            gemm_store(comm_al[h], orig_l * m_per + half)

        started = []

        def start(rds):
            for r in rds:
                r.start()
            started.extend(rds)
            return rds

        xc[0] = x_ref[pl.ds(0, half), :].astype(jnp.float8_e4m3fn)
        z0 = rdma(xc.at[0], zbuf.at[0], z_send.at[0], z_recv.at[0], zpartner)
        a0r = rdma(xc.at[0], comm_ar.at[0], ar_send.at[0], ar_recv.at[0], right)
        start([z0, a0r])
        xc[1] = x_ref[pl.ds(half, half), :].astype(jnp.float8_e4m3fn)
        z1 = rdma(xc.at[1], zbuf.at[1], z_send.at[1], z_recv.at[1], zpartner)
        a0l = rdma(xc.at[1], comm_al.at[0], al_send.at[0], al_recv.at[0], left)
        start([z1, a0l])

        wbc[...] = w_ref[...].astype(jnp.bfloat16)
        gemm_store(x_ref[...], my * m_per)

        a0r.wait_recv()
        a0l.wait_recv()
        a1 = start(a_hop(1))
        st_c2r[...] = comm_ar[0, pl.ds(0, DELTA), :]
        st_c2l[...] = comm_al[0, pl.ds(rest, DELTA), :]
        compute_a(0)

        z0.wait_recv()
        z1.wait_recv()
        st_br[...] = zbuf[0, pl.ds(DELTA, rest), :]
        st_bl[...] = zbuf[1, pl.ds(0, rest), :]
        b = start([
            rdma(st_br, br_a, br_send.at[0], br_recv.at[0], right),
            rdma(zbuf.at[1], br_b, br_send.at[1], br_recv.at[1], right),
            rdma(zbuf.at[0], bl_a, bl_send.at[0], bl_recv.at[0], left),
            rdma(st_bl, bl_b, bl_send.at[1], bl_recv.at[1], left),
        ])
        c2 = start([
            rdma(st_c2r, czb2r, c2_send.at[0], c2_recv.at[0], zpartner),
            rdma(st_c2l, czb2l, c2_send.at[1], c2_recv.at[1], zpartner),
        ])
        gemm_store(zbuf[0], zpartner * m_per)
        gemm_store(zbuf[1], zpartner * m_per + half)

        for rd in a1:
            rd.wait_recv()
        a2 = start(a_hop(2))
        c = start([
            rdma(comm_ar.at[1], czbuf.at[0], c_send.at[0], c_recv.at[0], zpartner),
            rdma(comm_al.at[1], czbuf.at[1], c_send.at[1], c_recv.at[1], zpartner),
        ])
        compute_a(1)

        orig_br = other + lax.rem(p + 3, N_PLANE)
        orig_bl = other + lax.rem(p + 1, N_PLANE)
        for rd in b:
            rd.wait_recv()
        gemm_store(br_a[...], orig_br * m_per + DELTA)
        gemm_store(br_b[...], orig_br * m_per + half)
        gemm_store(bl_a[...], orig_bl * m_per)
        gemm_store(bl_b[...], orig_bl * m_per + half)

        for rd in c2:
            rd.wait_recv()
        gemm_store(czb2r[...], orig_br * m_per)
        gemm_store(czb2l[...], orig_bl * m_per + half + rest)

        for rd in c:
            rd.wait_recv()
        orig_c = other + lax.rem(p + 2, N_PLANE)
        gemm_store(czbuf[0], orig_c * m_per)
        gemm_store(czbuf[1], orig_c * m_per + half)

        for rd in a2:
            rd.wait_recv()
        compute_a(2)

        for rd in started:
            rd.wait_send()

    f8 = jnp.float8_e4m3fn
    half_pair = lambda: pltpu.VMEM((2, half, k), f8)
    pair_sem = lambda: pltpu.SemaphoreType.DMA((2,))
    hop_sem = lambda: pltpu.SemaphoreType.DMA((N_HOP,))
    return pl.pallas_call(
        body,
        out_shape=jax.ShapeDtypeStruct((N_DEV * m_per, n_per), jnp.float32),
        in_specs=[
            pl.BlockSpec(memory_space=pltpu.VMEM),
            pl.BlockSpec(memory_space=pltpu.VMEM),
            pl.BlockSpec(memory_space=pltpu.SMEM),
        ],
        out_specs=pl.BlockSpec(memory_space=pltpu.VMEM),
        scratch_shapes=[
            half_pair(),
            pltpu.VMEM((k, n_per), jnp.bfloat16),
            half_pair(),
            pltpu.VMEM((N_HOP, half, k), f8),
            pltpu.VMEM((N_HOP, half, k), f8),
            pltpu.VMEM((rest, k), f8),
            pltpu.VMEM((half, k), f8),
            pltpu.VMEM((half, k), f8),
            pltpu.VMEM((rest, k), f8),
            half_pair(),
            pltpu.VMEM((DELTA, k), f8),
            pltpu.VMEM((DELTA, k), f8),
            pltpu.VMEM((rest, k), f8),
            pltpu.VMEM((rest, k), f8),
            pltpu.VMEM((DELTA, k), f8),
            pltpu.VMEM((DELTA, k), f8),
            pair_sem(), pair_sem(),
            hop_sem(), hop_sem(), hop_sem(), hop_sem(),
            pair_sem(), pair_sem(),
            pair_sem(), pair_sem(),
            pair_sem(), pair_sem(),
            pair_sem(), pair_sem(),
        ],
        compiler_params=pltpu.CompilerParams(collective_id=0),
    )(x, w_mat, s)
